# baseline (speedup 1.0000x reference)
import functools

import jax
import jax.numpy as jnp
from jax import lax
from jax.experimental import pallas as pl
from jax.experimental.pallas import tpu as pltpu
from jax.experimental.pallas import tpu_sc as plsc

_BATCH = 4096
_SEQ = 200
_EMB = 64
_NW = 32
_BC = 256
_NBC = _BATCH // _BC
_NSTEP = _SEQ * _NBC
_KSTEPS = _NSTEP // _NW

_mesh = plsc.VectorSubcoreMesh(core_axis_name="c", subcore_axis_name="s")


@functools.partial(
    pl.kernel,
    mesh=_mesh,
    out_type=jax.ShapeDtypeStruct((_SEQ, 8, 32, 1024), jnp.float32),
    scratch_types=[
        pltpu.VMEM((2, 2, 8, 128), jnp.int32),
        pltpu.VMEM((2, 2, 128), jnp.int32),
        pltpu.VMEM((2, _BC, _EMB), jnp.float32),
        pltpu.VMEM((2, 8, 2, 1024), jnp.float32),
        pltpu.SemaphoreType.DMA((2,)),
        pltpu.SemaphoreType.DMA((2,)),
    ],
    compiler_params=pltpu.CompilerParams(
        use_tc_tiling_on_sc=False, needs_layout_passes=False
    ),
)
def _embed_kernel(ids_hbm, tab_hbm, out_hbm, idsb, idxp, rows, outt, gsem, osem):
    wid = lax.axis_index("s") * 2 + lax.axis_index("c")
    iota = lax.iota(jnp.int32, 16)

    def step_sc(k):
        g = wid + k * _NW
        return g // _NBC, g % _NBC

    def start_step(k, b):
        s, bc = step_sc(k)
        sb, sr = s // 8, s % 8
        pltpu.sync_copy(ids_hbm.at[sb, pl.ds(2 * bc, 2)], idsb.at[b])
        for tj in range(2):
            for q in range(8):
                v = idsb[b, tj, sr, pl.ds(16 * q, 16)]
                idxp[b, tj, pl.ds(16 * q, 16)] = lax.shift_left(v, 1)
        for j in range(2):
            pltpu.async_copy(
                tab_hbm.at[idxp.at[b, j]],
                rows.at[b, pl.ds(128 * j, 128)],
                gsem.at[b],
            )

    def gather_wait(b):
        for j in range(2):
            pltpu.make_async_copy(
                tab_hbm.at[idxp.at[b, j]],
                rows.at[b, pl.ds(128 * j, 128)],
                gsem.at[b],
            ).wait()

    def transpose_store(k, b):
        s, bc = step_sc(k)
        rrows = []
        wtj = []
        wc = []
        for g in range(16):
            rrows.append(iota + 16 * g)
            wtj.append(jnp.full((16,), g // 8, jnp.int32))
            wc.append(iota + 16 * (g % 8))

        def ebody(e, carry):
            d = jnp.bitwise_and(e + iota, _EMB - 1)
            ti = lax.shift_right_logical(d, 3)
            r128 = lax.shift_left(jnp.bitwise_and(d, 7), 7)
            for g in range(16):
                vals = plsc.load_gather(rows.at[b], [rrows[g], d])
                plsc.store_scatter(
                    outt.at[b], [ti, wtj[g], r128 + wc[g]], vals
                )
            return carry

        lax.fori_loop(0, _EMB, ebody, 0, unroll=4)
        pltpu.async_copy(
            outt.at[b], out_hbm.at[s, :, pl.ds(2 * bc, 2)], osem.at[b]
        )

    def store_wait(k, b):
        s, bc = step_sc(k)
        pltpu.make_async_copy(
            outt.at[b], out_hbm.at[s, :, pl.ds(2 * bc, 2)], osem.at[b]
        ).wait()

    start_step(0, 0)

    def body(t, carry):
        for b in range(2):
            k = t * 2 + b

            @pl.when(k + 1 < _KSTEPS)
            def _():
                start_step(k + 1, 1 - b)

            gather_wait(b)

            @pl.when(k >= 2)
            def _():
                store_wait(k - 2, b)

            transpose_store(k, b)
        return carry

    lax.fori_loop(0, _KSTEPS // 2, body, 0)
    store_wait(_KSTEPS - 2, 0)
    store_wait(_KSTEPS - 1, 1)


def kernel(prompt_token_ids, embedding_table):
    ids5 = jnp.transpose(
        prompt_token_ids.T.reshape(25, 8, 32, 128), (0, 2, 1, 3)
    )
    tab = jnp.pad(embedding_table, ((0, 0), (0, 64))).reshape(2000000, _EMB)
    out5 = _embed_kernel(ids5, tab).reshape(_SEQ, 8, 32, 8, 128)
    out = jnp.transpose(out5, (2, 4, 0, 1, 3))
    return out.reshape(_BATCH, _SEQ, _EMB)


# speedup vs baseline: 1.2055x; 1.2055x over previous
import functools

import jax
import jax.numpy as jnp
from jax import lax
from jax.experimental import pallas as pl
from jax.experimental.pallas import tpu as pltpu
from jax.experimental.pallas import tpu_sc as plsc

_BATCH = 4096
_SEQ = 200
_EMB = 64
_TOTAL = _BATCH * _SEQ
_NW = 32
_B_PER_W = _TOTAL // _NW
_CHUNK = 640
_NCHUNK = _B_PER_W // _CHUNK
_NBUF = 2
_KSUB = 8
_SUB = _CHUNK // _KSUB

_mesh = plsc.VectorSubcoreMesh(core_axis_name="c", subcore_axis_name="s")


@functools.partial(
    pl.kernel,
    mesh=_mesh,
    out_type=jax.ShapeDtypeStruct((_TOTAL, 128), jnp.float32),
    scratch_types=[
        pltpu.VMEM((_B_PER_W,), jnp.int32),
        pltpu.VMEM((_NBUF, _CHUNK, _EMB), jnp.float32),
        pltpu.SemaphoreType.DMA((_NBUF,)),
        pltpu.SemaphoreType.DMA((_NBUF,)),
    ],
    compiler_params=pltpu.CompilerParams(use_tc_tiling_on_sc=False),
)
def _gather_kernel(ids_hbm, table_hbm, out_hbm, idx_v, rows_v, gsem, osem):
    wid = lax.axis_index("s") * 2 + lax.axis_index("c")
    base = wid * _B_PER_W

    pltpu.sync_copy(ids_hbm.at[pl.ds(base, _B_PER_W)], idx_v)

    def dbl(q, c):
        idx_v[pl.ds(16 * q, 16)] = lax.shift_left(
            idx_v[pl.ds(16 * q, 16)], 1
        )
        return c

    lax.fori_loop(0, _B_PER_W // 16, dbl, 0, unroll=8)

    def gather_start(j, b):
        for k in range(_KSUB):
            idx = idx_v.at[pl.ds(j * _CHUNK + k * _SUB, _SUB)]
            dst = rows_v.at[b, pl.ds(k * _SUB, _SUB)]
            pltpu.async_copy(table_hbm.at[idx], dst, gsem.at[b])

    def gather_wait(j, b):
        for k in range(_KSUB):
            idx = idx_v.at[pl.ds(j * _CHUNK + k * _SUB, _SUB)]
            dst = rows_v.at[b, pl.ds(k * _SUB, _SUB)]
            pltpu.make_async_copy(table_hbm.at[idx], dst, gsem.at[b]).wait()

    def store_start(j, b):
        out = out_hbm.at[pl.ds(base + j * _CHUNK, _CHUNK), pl.ds(0, _EMB)]
        pltpu.async_copy(rows_v.at[b], out, osem.at[b])

    def store_wait(j, b):
        out = out_hbm.at[pl.ds(base + j * _CHUNK, _CHUNK), pl.ds(0, _EMB)]
        pltpu.make_async_copy(rows_v.at[b], out, osem.at[b]).wait()

    for b in range(_NBUF):
        gather_start(b, b)

    def body(t, carry):
        for b in range(_NBUF):
            j = t * _NBUF + b
            gather_wait(j, b)
            store_start(j, b)

            @pl.when(j < _NCHUNK - _NBUF)
            def _():
                store_wait(j, b)
                gather_start(j + _NBUF, b)

        return carry

    lax.fori_loop(0, _NCHUNK // _NBUF, body, 0)

    for b in range(_NBUF):
        store_wait(_NCHUNK - _NBUF + b, b)


def kernel(prompt_token_ids, embedding_table):
    ids = prompt_token_ids.reshape(_TOTAL)
    tab = jnp.pad(embedding_table, ((0, 0), (0, 64))).reshape(2 * 1000000, _EMB)
    out = _gather_kernel(ids, tab)
    return out[:, :_EMB].reshape(_BATCH, _SEQ, _EMB)
